# baseline (speedup 1.0000x reference)
import functools

import jax
import jax.numpy as jnp
from jax import lax
from jax.experimental import pallas as pl
from jax.experimental.pallas import tpu as pltpu
from jax.experimental.pallas import tpu_sc as plsc

B = 16384
F = 26
V = 1_040_000

NC = 2
NS = 16
NW = NC * NS
BPW = B // NW
CB = 128
NCHUNK = BPW // CB
NROWS = F * NCHUNK
FIRE = 8
NGRP = NROWS // FIRE


def _sc_lookup_sum(x_r, table, bias16):
    mesh = plsc.VectorSubcoreMesh(core_axis_name="c", subcore_axis_name="s")

    @functools.partial(
        pl.kernel,
        mesh=mesh,
        out_type=jax.ShapeDtypeStruct((NW, NCHUNK, CB), jnp.float32),
        scratch_types=[
            pltpu.VMEM((NROWS * CB,), jnp.int32),
            pltpu.VMEM((NROWS * CB,), jnp.float32),
            pltpu.VMEM((NCHUNK, CB), jnp.float32),
            pltpu.VMEM((16,), jnp.float32),
            pltpu.VMEM_SHARED((V,), jnp.float32),
            pltpu.VMEM((13000,), jnp.float32),
            pltpu.VMEM((13000,), jnp.float32),
            pltpu.SemaphoreType.DMA,
            pltpu.SemaphoreType.DMA,
        ],
    )
    def k(x_hbm, table_hbm, bias_hbm, out_hbm, idx_v, vals_v, out_v, bias_v,
          table_sh, stage_a, stage_b, sem, sem2):
        sid = lax.axis_index("s")
        wid = sid * NC + lax.axis_index("c")
        vs = V // NS
        SCH = 13000
        nst = vs // SCH
        pltpu.sync_copy(bias_hbm, bias_v)
        stages = [stage_a, stage_b]
        out_cps = [None, None]
        for j in range(nst):
            off = sid * vs + j * SCH
            if out_cps[j % 2] is not None:
                out_cps[j % 2].wait()
            pltpu.async_copy(
                table_hbm.at[pl.ds(off, SCH)], stages[j % 2], sem
            ).wait()
            out_cps[j % 2] = pltpu.async_copy(
                stages[j % 2], table_sh.at[pl.ds(off, SCH)], sem2
            )
        pltpu.sync_copy(x_hbm.at[wid], idx_v)
        for cp in out_cps:
            cp.wait()
        plsc.subcore_barrier()

        HSPLIT = 5992
        cp_h = pltpu.async_copy(
            table_hbm.at[idx_v.at[pl.ds(0, HSPLIT)]],
            vals_v.at[pl.ds(0, HSPLIT)], sem,
        )
        cp_s = pltpu.async_copy(
            table_sh.at[idx_v.at[pl.ds(HSPLIT, NROWS * CB - HSPLIT)]],
            vals_v.at[pl.ds(HSPLIT, NROWS * CB - HSPLIT)], sem2,
        )
        cp_h.wait()
        cp_s.wait()

        bv = bias_v[...]
        for c in range(NCHUNK):
            for lb in range(CB // 16):
                sl = pl.ds(lb * 16, 16)
                acc = bv
                for f in range(F):
                    acc = acc + vals_v[pl.ds((f * NCHUNK + c) * CB + lb * 16, 16)]
                out_v[c, sl] = acc

        pltpu.sync_copy(out_v, out_hbm.at[wid])

    return k(x_r, table, bias16)


def kernel(x, fc_weight, bias):
    x_r = (
        x.reshape(NW, NCHUNK, CB, F)
        .transpose(0, 3, 1, 2)
        .reshape(NW, NROWS * CB)
    )
    table = fc_weight.reshape(V)
    bias16 = jnp.broadcast_to(bias.astype(jnp.float32), (16,))
    out = _sc_lookup_sum(x_r, table, bias16)
    return out.reshape(B, 1)


# speedup vs baseline: 1.0453x; 1.0453x over previous
import functools

import jax
import jax.numpy as jnp
from jax import lax
from jax.experimental import pallas as pl
from jax.experimental.pallas import tpu as pltpu
from jax.experimental.pallas import tpu_sc as plsc

B = 16384
F = 26
V = 1_040_000

NC = 2
NS = 16
NW = NC * NS
BPW = B // NW
CB = 128
NCHUNK = BPW // CB
NROWS = F * NCHUNK
FIRE = 8
NGRP = NROWS // FIRE


def _sc_lookup_sum(x_r, table, bias16):
    mesh = plsc.VectorSubcoreMesh(core_axis_name="c", subcore_axis_name="s")

    @functools.partial(
        pl.kernel,
        mesh=mesh,
        out_type=jax.ShapeDtypeStruct((NW, NCHUNK, CB), jnp.float32),
        scratch_types=[
            pltpu.VMEM((NROWS * CB,), jnp.int32),
            pltpu.VMEM((NROWS * CB,), jnp.float32),
            pltpu.VMEM((NCHUNK, CB), jnp.float32),
            pltpu.VMEM((16,), jnp.float32),
            pltpu.VMEM_SHARED((V,), jnp.float32),
            pltpu.VMEM((13000,), jnp.float32),
            pltpu.VMEM((13000,), jnp.float32),
            pltpu.SemaphoreType.DMA,
            pltpu.SemaphoreType.DMA,
        ],
    )
    def k(x_hbm, table_hbm, bias_hbm, out_hbm, idx_v, vals_v, out_v, bias_v,
          table_sh, stage_a, stage_b, sem, sem2):
        sid = lax.axis_index("s")
        wid = sid * NC + lax.axis_index("c")
        vs = V // NS
        SCH = 13000
        nst = vs // SCH
        pltpu.sync_copy(bias_hbm, bias_v)
        stages = [stage_a, stage_b]
        out_cps = [None, None]
        for j in range(nst):
            off = sid * vs + j * SCH
            if out_cps[j % 2] is not None:
                out_cps[j % 2].wait()
            pltpu.async_copy(
                table_hbm.at[pl.ds(off, SCH)], stages[j % 2], sem
            ).wait()
            out_cps[j % 2] = pltpu.async_copy(
                stages[j % 2], table_sh.at[pl.ds(off, SCH)], sem2
            )
        pltpu.sync_copy(x_hbm.at[wid], idx_v)
        for cp in out_cps:
            cp.wait()
        plsc.subcore_barrier()

        pltpu.async_copy(table_sh.at[idx_v], vals_v, sem).wait()

        bv = bias_v[...]
        for c in range(NCHUNK):
            for lb in range(CB // 16):
                sl = pl.ds(lb * 16, 16)
                acc = bv
                for f in range(F):
                    acc = acc + vals_v[pl.ds((f * NCHUNK + c) * CB + lb * 16, 16)]
                out_v[c, sl] = acc

        pltpu.sync_copy(out_v, out_hbm.at[wid])

    return k(x_r, table, bias16)


def kernel(x, fc_weight, bias):
    x_r = (
        x.reshape(NW, NCHUNK, CB, F)
        .transpose(0, 3, 1, 2)
        .reshape(NW, NROWS * CB)
    )
    table = fc_weight.reshape(V)
    bias16 = jnp.broadcast_to(bias.astype(jnp.float32), (16,))
    out = _sc_lookup_sum(x_r, table, bias16)
    return out.reshape(B, 1)
